# baseline (speedup 1.0000x reference)
import functools

import jax
import jax.numpy as jnp
from jax import lax
from jax.experimental import pallas as pl
from jax.experimental.pallas import tpu as pltpu
from jax.experimental.pallas import tpu_sc as plsc

NUM_ENTITIES = 1000000
NUM_RELATIONS = 1000
D = 128
HALF = 64
BATCH = 16384

NC = 2
NS = 16
NW = NC * NS
B_PER_W = BATCH // NW
CHUNK = 128
N_CHUNKS = B_PER_W // CHUNK


def _lane_perm(x, idx):
    dn = lax.GatherDimensionNumbers(
        offset_dims=(), collapsed_slice_dims=(0,), start_index_map=(0,))
    return lax.gather(x, idx[:, None], dn, (1,),
                      mode=lax.GatherScatterMode.PROMISE_IN_BOUNDS)


def _score_body(ent_hbm, rel_hbm, head_hbm, ridx_hbm, tail_hbm, out_hbm,
                idx_h, idx_r, idx_t, rh0, rr0, rt0, rh1, rr1, rt1,
                out_v, sem0, sem1):
    wid = lax.axis_index("s") * NC + lax.axis_index("c")
    base = wid * B_PER_W
    lane = jax.lax.iota(jnp.int32, 16)

    pltpu.sync_copy(head_hbm.at[pl.ds(base, B_PER_W)], idx_h)
    pltpu.sync_copy(ridx_hbm.at[pl.ds(base, B_PER_W)], idx_r)
    pltpu.sync_copy(tail_hbm.at[pl.ds(base, B_PER_W)], idx_t)

    bufs = ((rh0, rr0, rt0, sem0), (rh1, rr1, rt1, sem1))

    def start(c):
        rh, rr, rt, sem = bufs[c % 2]
        s = pl.ds(c * CHUNK, CHUNK)
        return (pltpu.async_copy(ent_hbm.at[idx_h.at[s]], rh, sem),
                pltpu.async_copy(rel_hbm.at[idx_r.at[s]], rr, sem),
                pltpu.async_copy(ent_hbm.at[idx_t.at[s]], rt, sem))

    def compute(c):
        rows_h, rows_r, rows_t, _ = bufs[c % 2]

        @plsc.parallel_loop(0, CHUNK, 1, unroll=4)
        def elem(i):
            ms = []
            for g in range(HALF // 16):
                lo = g * 16
                hr = rows_h[i, pl.ds(lo, 16)]
                hi = rows_h[i, pl.ds(HALF + lo, 16)]
                rr = rows_r[i, pl.ds(lo, 16)]
                ri = rows_r[i, pl.ds(HALF + lo, 16)]
                tr = rows_t[i, pl.ds(lo, 16)]
                ti = rows_t[i, pl.ds(HALF + lo, 16)]
                ms.append(hr * (rr * tr + ri * ti) + hi * (rr * ti - ri * tr))
            acc = (ms[0] + ms[1]) + (ms[2] + ms[3])
            for sh in (8, 4, 2, 1):
                acc = acc + _lane_perm(acc, jnp.bitwise_xor(lane, sh))
            onehot = jnp.where(lane == jnp.bitwise_and(i, 15), acc, 0.0)
            slot = pl.multiple_of(c * CHUNK + jnp.bitwise_and(i, -16), 16)
            plsc.addupdate(out_v.at[pl.ds(slot, 16)], onehot)

    cps = start(0)
    zeros16 = jnp.zeros((16,), jnp.float32)
    for z in range(B_PER_W // 16):
        out_v[pl.ds(z * 16, 16)] = zeros16
    for c in range(N_CHUNKS):
        nxt = start(c + 1) if c + 1 < N_CHUNKS else None
        for cp in cps:
            cp.wait()
        compute(c)
        cps = nxt
    pltpu.sync_copy(out_v, out_hbm.at[pl.ds(base, B_PER_W)])


@jax.jit
def _complex_score(entity_weight, relation_weight, head, relation, tail):
    mesh = plsc.VectorSubcoreMesh(core_axis_name="c", subcore_axis_name="s")
    k = functools.partial(
        pl.kernel,
        out_type=jax.ShapeDtypeStruct((BATCH,), jnp.float32),
        mesh=mesh,
        scratch_types=[
            pltpu.VMEM((B_PER_W,), jnp.int32),
            pltpu.VMEM((B_PER_W,), jnp.int32),
            pltpu.VMEM((B_PER_W,), jnp.int32),
            pltpu.VMEM((CHUNK, D), jnp.float32),
            pltpu.VMEM((CHUNK, D), jnp.float32),
            pltpu.VMEM((CHUNK, D), jnp.float32),
            pltpu.VMEM((CHUNK, D), jnp.float32),
            pltpu.VMEM((CHUNK, D), jnp.float32),
            pltpu.VMEM((CHUNK, D), jnp.float32),
            pltpu.VMEM((B_PER_W,), jnp.float32),
            pltpu.SemaphoreType.DMA,
            pltpu.SemaphoreType.DMA,
        ],
    )(_score_body)
    return k(entity_weight, relation_weight, head, relation, tail)


def kernel(entity_weight, relation_weight, head, relation, tail):
    return _complex_score(
        entity_weight,
        relation_weight,
        head.astype(jnp.int32),
        relation.astype(jnp.int32),
        tail.astype(jnp.int32),
    )


# speedup vs baseline: 1.0640x; 1.0640x over previous
import functools

import jax
import jax.numpy as jnp
from jax import lax
from jax.experimental import pallas as pl
from jax.experimental.pallas import tpu as pltpu
from jax.experimental.pallas import tpu_sc as plsc

NUM_ENTITIES = 1000000
NUM_RELATIONS = 1000
D = 128
HALF = 64
BATCH = 16384

NC = 2
NS = 16
NW = NC * NS
B_PER_W = BATCH // NW
CHUNK = 128
N_CHUNKS = B_PER_W // CHUNK


def _lane_perm(x, idx):
    dn = lax.GatherDimensionNumbers(
        offset_dims=(), collapsed_slice_dims=(0,), start_index_map=(0,))
    return lax.gather(x, idx[:, None], dn, (1,),
                      mode=lax.GatherScatterMode.PROMISE_IN_BOUNDS)


def _score_body(ent_hbm, rel_hbm, head_hbm, ridx_hbm, tail_hbm, out_hbm,
                idx_h, idx_r, idx_t, rh0, rr0, rt0, rh1, rr1, rt1,
                out_v, sem0, sem1):
    wid = lax.axis_index("s") * NC + lax.axis_index("c")
    base = wid * B_PER_W
    lane = jax.lax.iota(jnp.int32, 16)

    pltpu.sync_copy(head_hbm.at[pl.ds(base, B_PER_W)], idx_h)
    pltpu.sync_copy(ridx_hbm.at[pl.ds(base, B_PER_W)], idx_r)
    pltpu.sync_copy(tail_hbm.at[pl.ds(base, B_PER_W)], idx_t)

    bufs = ((rh0, rr0, rt0, sem0), (rh1, rr1, rt1, sem1))

    def start(c):
        rh, rr, rt, sem = bufs[c % 2]
        s = pl.ds(c * CHUNK, CHUNK)
        return (pltpu.async_copy(ent_hbm.at[idx_h.at[s]], rh, sem),
                pltpu.async_copy(rel_hbm.at[idx_r.at[s]], rr, sem),
                pltpu.async_copy(ent_hbm.at[idx_t.at[s]], rt, sem))

    def compute(c):
        rows_h, rows_r, rows_t, _ = bufs[c % 2]

        @plsc.parallel_loop(0, CHUNK, 1, unroll=2)
        def elem(i):
            ms = []
            for g in range(HALF // 16):
                lo = g * 16
                hr = rows_h[i, pl.ds(lo, 16)]
                hi = rows_h[i, pl.ds(HALF + lo, 16)]
                rr = rows_r[i, pl.ds(lo, 16)]
                ri = rows_r[i, pl.ds(HALF + lo, 16)]
                tr = rows_t[i, pl.ds(lo, 16)]
                ti = rows_t[i, pl.ds(HALF + lo, 16)]
                ms.append(hr * (rr * tr + ri * ti) + hi * (rr * ti - ri * tr))
            acc = (ms[0] + ms[1]) + (ms[2] + ms[3])
            for sh in (8, 4, 2, 1):
                acc = acc + _lane_perm(acc, jnp.bitwise_xor(lane, sh))
            onehot = jnp.where(lane == jnp.bitwise_and(i, 15), acc, 0.0)
            slot = pl.multiple_of(c * CHUNK + jnp.bitwise_and(i, -16), 16)
            plsc.addupdate(out_v.at[pl.ds(slot, 16)], onehot)

    cps = start(0)
    zeros16 = jnp.zeros((16,), jnp.float32)
    for z in range(B_PER_W // 16):
        out_v[pl.ds(z * 16, 16)] = zeros16
    for c in range(N_CHUNKS):
        nxt = start(c + 1) if c + 1 < N_CHUNKS else None
        for cp in cps:
            cp.wait()
        compute(c)
        cps = nxt
    pltpu.sync_copy(out_v, out_hbm.at[pl.ds(base, B_PER_W)])


@jax.jit
def _complex_score(entity_weight, relation_weight, head, relation, tail):
    mesh = plsc.VectorSubcoreMesh(core_axis_name="c", subcore_axis_name="s")
    k = functools.partial(
        pl.kernel,
        out_type=jax.ShapeDtypeStruct((BATCH,), jnp.float32),
        mesh=mesh,
        scratch_types=[
            pltpu.VMEM((B_PER_W,), jnp.int32),
            pltpu.VMEM((B_PER_W,), jnp.int32),
            pltpu.VMEM((B_PER_W,), jnp.int32),
            pltpu.VMEM((CHUNK, D), jnp.float32),
            pltpu.VMEM((CHUNK, D), jnp.float32),
            pltpu.VMEM((CHUNK, D), jnp.float32),
            pltpu.VMEM((CHUNK, D), jnp.float32),
            pltpu.VMEM((CHUNK, D), jnp.float32),
            pltpu.VMEM((CHUNK, D), jnp.float32),
            pltpu.VMEM((B_PER_W,), jnp.float32),
            pltpu.SemaphoreType.DMA,
            pltpu.SemaphoreType.DMA,
        ],
    )(_score_body)
    return k(entity_weight, relation_weight, head, relation, tail)


def kernel(entity_weight, relation_weight, head, relation, tail):
    return _complex_score(
        entity_weight,
        relation_weight,
        head.astype(jnp.int32),
        relation.astype(jnp.int32),
        tail.astype(jnp.int32),
    )


# speedup vs baseline: 1.0799x; 1.0149x over previous
import functools

import jax
import jax.numpy as jnp
from jax import lax
from jax.experimental import pallas as pl
from jax.experimental.pallas import tpu as pltpu
from jax.experimental.pallas import tpu_sc as plsc

NUM_ENTITIES = 1000000
NUM_RELATIONS = 1000
D = 128
HALF = 64
BATCH = 16384

NC = 2
NS = 16
NW = NC * NS
B_PER_W = BATCH // NW
CHUNK = 64
N_CHUNKS = B_PER_W // CHUNK
NBUF = 3


def _lane_perm(x, idx):
    dn = lax.GatherDimensionNumbers(
        offset_dims=(), collapsed_slice_dims=(0,), start_index_map=(0,))
    return lax.gather(x, idx[:, None], dn, (1,),
                      mode=lax.GatherScatterMode.PROMISE_IN_BOUNDS)


def _score_body(ent_hbm, rel_hbm, head_hbm, ridx_hbm, tail_hbm, out_hbm,
                idx_h, idx_r, idx_t, rh0, rr0, rt0, rh1, rr1, rt1,
                rh2, rr2, rt2, out_v, sem0, sem1, sem2):
    wid = lax.axis_index("s") * NC + lax.axis_index("c")
    base = wid * B_PER_W
    lane = jax.lax.iota(jnp.int32, 16)

    pltpu.sync_copy(head_hbm.at[pl.ds(base, B_PER_W)], idx_h)
    pltpu.sync_copy(ridx_hbm.at[pl.ds(base, B_PER_W)], idx_r)
    pltpu.sync_copy(tail_hbm.at[pl.ds(base, B_PER_W)], idx_t)

    bufs = ((rh0, rr0, rt0, sem0), (rh1, rr1, rt1, sem1),
            (rh2, rr2, rt2, sem2))

    def start(c):
        rh, rr, rt, sem = bufs[c % NBUF]
        s = pl.ds(c * CHUNK, CHUNK)
        return (pltpu.async_copy(ent_hbm.at[idx_h.at[s]], rh, sem),
                pltpu.async_copy(rel_hbm.at[idx_r.at[s]], rr, sem),
                pltpu.async_copy(ent_hbm.at[idx_t.at[s]], rt, sem))

    def compute(c):
        rows_h, rows_r, rows_t, _ = bufs[c % NBUF]

        @plsc.parallel_loop(0, CHUNK, 1, unroll=2)
        def elem(i):
            ms = []
            for g in range(HALF // 16):
                lo = g * 16
                hr = rows_h[i, pl.ds(lo, 16)]
                hi = rows_h[i, pl.ds(HALF + lo, 16)]
                rr = rows_r[i, pl.ds(lo, 16)]
                ri = rows_r[i, pl.ds(HALF + lo, 16)]
                tr = rows_t[i, pl.ds(lo, 16)]
                ti = rows_t[i, pl.ds(HALF + lo, 16)]
                ms.append(hr * (rr * tr + ri * ti) + hi * (rr * ti - ri * tr))
            acc = (ms[0] + ms[1]) + (ms[2] + ms[3])
            for sh in (8, 4, 2, 1):
                acc = acc + _lane_perm(acc, jnp.bitwise_xor(lane, sh))
            onehot = jnp.where(lane == jnp.bitwise_and(i, 15), acc, 0.0)
            slot = pl.multiple_of(c * CHUNK + jnp.bitwise_and(i, -16), 16)
            plsc.addupdate(out_v.at[pl.ds(slot, 16)], onehot)

    pending = [start(c) for c in range(NBUF - 1)]
    zeros16 = jnp.zeros((16,), jnp.float32)
    for z in range(B_PER_W // 16):
        out_v[pl.ds(z * 16, 16)] = zeros16
    for c in range(N_CHUNKS):
        if c + NBUF - 1 < N_CHUNKS:
            pending.append(start(c + NBUF - 1))
        for cp in pending.pop(0):
            cp.wait()
        compute(c)
    pltpu.sync_copy(out_v, out_hbm.at[pl.ds(base, B_PER_W)])


@jax.jit
def _complex_score(entity_weight, relation_weight, head, relation, tail):
    mesh = plsc.VectorSubcoreMesh(core_axis_name="c", subcore_axis_name="s")
    k = functools.partial(
        pl.kernel,
        out_type=jax.ShapeDtypeStruct((BATCH,), jnp.float32),
        mesh=mesh,
        scratch_types=[
            pltpu.VMEM((B_PER_W,), jnp.int32),
            pltpu.VMEM((B_PER_W,), jnp.int32),
            pltpu.VMEM((B_PER_W,), jnp.int32),
            pltpu.VMEM((CHUNK, D), jnp.float32),
            pltpu.VMEM((CHUNK, D), jnp.float32),
            pltpu.VMEM((CHUNK, D), jnp.float32),
            pltpu.VMEM((CHUNK, D), jnp.float32),
            pltpu.VMEM((CHUNK, D), jnp.float32),
            pltpu.VMEM((CHUNK, D), jnp.float32),
            pltpu.VMEM((CHUNK, D), jnp.float32),
            pltpu.VMEM((CHUNK, D), jnp.float32),
            pltpu.VMEM((CHUNK, D), jnp.float32),
            pltpu.VMEM((B_PER_W,), jnp.float32),
            pltpu.SemaphoreType.DMA,
            pltpu.SemaphoreType.DMA,
            pltpu.SemaphoreType.DMA,
        ],
    )(_score_body)
    return k(entity_weight, relation_weight, head, relation, tail)


def kernel(entity_weight, relation_weight, head, relation, tail):
    return _complex_score(
        entity_weight,
        relation_weight,
        head.astype(jnp.int32),
        relation.astype(jnp.int32),
        tail.astype(jnp.int32),
    )


# speedup vs baseline: 1.1112x; 1.0290x over previous
import functools

import jax
import jax.numpy as jnp
from jax import lax
from jax.experimental import pallas as pl
from jax.experimental.pallas import tpu as pltpu
from jax.experimental.pallas import tpu_sc as plsc

NUM_ENTITIES = 1000000
NUM_RELATIONS = 1000
D = 128
HALF = 64
BATCH = 16384

NC = 2
NS = 16
NW = NC * NS
B_PER_W = BATCH // NW
CHUNK = 64
N_CHUNKS = B_PER_W // CHUNK
NBUF = 3


def _lane_perm(x, idx):
    dn = lax.GatherDimensionNumbers(
        offset_dims=(), collapsed_slice_dims=(0,), start_index_map=(0,))
    return lax.gather(x, idx[:, None], dn, (1,),
                      mode=lax.GatherScatterMode.PROMISE_IN_BOUNDS)


def _score_body(ent_hbm, rel_hbm, head_hbm, ridx_hbm, tail_hbm, out_hbm,
                idx_h, idx_r, idx_t, rh0, rr0, rt0, rh1, rr1, rt1,
                rh2, rr2, rt2, out_v, sem0, sem1, sem2):
    wid = lax.axis_index("s") * NC + lax.axis_index("c")
    base = wid * B_PER_W
    lane = jax.lax.iota(jnp.int32, 16)

    bufs = ((rh0, rr0, rt0, sem0), (rh1, rr1, rt1, sem1),
            (rh2, rr2, rt2, sem2))

    def start(c):
        rh, rr, rt, sem = bufs[c % NBUF]
        s = pl.ds(c * CHUNK, CHUNK)
        return (pltpu.async_copy(ent_hbm.at[idx_h.at[s]], rh, sem),
                pltpu.async_copy(rel_hbm.at[idx_r.at[s]], rr, sem),
                pltpu.async_copy(ent_hbm.at[idx_t.at[s]], rt, sem))

    def compute(c):
        rows_h, rows_r, rows_t, _ = bufs[c % NBUF]

        @plsc.parallel_loop(0, CHUNK, 1, unroll=2)
        def elem(i):
            ms = []
            for g in range(HALF // 16):
                lo = g * 16
                hr = rows_h[i, pl.ds(lo, 16)]
                hi = rows_h[i, pl.ds(HALF + lo, 16)]
                rr = rows_r[i, pl.ds(lo, 16)]
                ri = rows_r[i, pl.ds(HALF + lo, 16)]
                tr = rows_t[i, pl.ds(lo, 16)]
                ti = rows_t[i, pl.ds(HALF + lo, 16)]
                ms.append(hr * (rr * tr + ri * ti) + hi * (rr * ti - ri * tr))
            acc = (ms[0] + ms[1]) + (ms[2] + ms[3])
            for sh in (8, 4, 2, 1):
                acc = acc + _lane_perm(acc, jnp.bitwise_xor(lane, sh))
            onehot = jnp.where(lane == jnp.bitwise_and(i, 15), acc, 0.0)
            slot = pl.multiple_of(c * CHUNK + jnp.bitwise_and(i, -16), 16)
            plsc.addupdate(out_v.at[pl.ds(slot, 16)], onehot)

    cpi = (pltpu.async_copy(head_hbm.at[pl.ds(base, B_PER_W)], idx_h, sem0),
           pltpu.async_copy(ridx_hbm.at[pl.ds(base, B_PER_W)], idx_r, sem0),
           pltpu.async_copy(tail_hbm.at[pl.ds(base, B_PER_W)], idx_t, sem0))
    zeros16 = jnp.zeros((16,), jnp.float32)
    for z in range(B_PER_W // 16):
        out_v[pl.ds(z * 16, 16)] = zeros16
    for cp in cpi:
        cp.wait()
    pending = [start(c) for c in range(NBUF - 1)]
    for c in range(N_CHUNKS):
        if c + NBUF - 1 < N_CHUNKS:
            pending.append(start(c + NBUF - 1))
        for cp in pending.pop(0):
            cp.wait()
        compute(c)
    pltpu.sync_copy(out_v, out_hbm.at[pl.ds(base, B_PER_W)])


@jax.jit
def _complex_score(entity_weight, relation_weight, head, relation, tail):
    mesh = plsc.VectorSubcoreMesh(core_axis_name="c", subcore_axis_name="s")
    k = functools.partial(
        pl.kernel,
        out_type=jax.ShapeDtypeStruct((BATCH,), jnp.float32),
        mesh=mesh,
        scratch_types=[
            pltpu.VMEM((B_PER_W,), jnp.int32),
            pltpu.VMEM((B_PER_W,), jnp.int32),
            pltpu.VMEM((B_PER_W,), jnp.int32),
            pltpu.VMEM((CHUNK, D), jnp.float32),
            pltpu.VMEM((CHUNK, D), jnp.float32),
            pltpu.VMEM((CHUNK, D), jnp.float32),
            pltpu.VMEM((CHUNK, D), jnp.float32),
            pltpu.VMEM((CHUNK, D), jnp.float32),
            pltpu.VMEM((CHUNK, D), jnp.float32),
            pltpu.VMEM((CHUNK, D), jnp.float32),
            pltpu.VMEM((CHUNK, D), jnp.float32),
            pltpu.VMEM((CHUNK, D), jnp.float32),
            pltpu.VMEM((B_PER_W,), jnp.float32),
            pltpu.SemaphoreType.DMA,
            pltpu.SemaphoreType.DMA,
            pltpu.SemaphoreType.DMA,
        ],
    )(_score_body)
    return k(entity_weight, relation_weight, head, relation, tail)


def kernel(entity_weight, relation_weight, head, relation, tail):
    return _complex_score(
        entity_weight,
        relation_weight,
        head.astype(jnp.int32),
        relation.astype(jnp.int32),
        tail.astype(jnp.int32),
    )
